# baseline (speedup 1.0000x reference)
import functools

import numpy as np

import jax
import jax.numpy as jnp
from jax.experimental import pallas as pl
from jax.experimental.pallas import tpu as pltpu


def _swish16(x):
    half = jnp.bfloat16(0.5)
    return x * (half * jnp.tanh(half * x) + half)


def _fused_net_kernel(cols_ref, ws_ref, bs_ref, we_ref, be_ref,
                      wdw_ref, bdw_ref, w1_ref, b1_ref, w2_ref, b2_ref,
                      wp_ref, bp_ref, wh_ref, bh_ref, wf_ref, bf_ref,
                      o_ref, ep_ref, *, Ho, Wo):
    S = Ho * Wo
    inv_s = 1.0 / S

    @pl.when(pl.program_id(0) == 0)
    def _():
        ep_ref[...] = jnp.zeros_like(ep_ref)

    cols = cols_ref[0]
    h = jnp.dot(cols, ws_ref[...],
                preferred_element_type=jnp.float32) + bs_ref[...]
    hb = _swish16(h.astype(jnp.bfloat16))

    e = jnp.dot(hb, we_ref[...],
                preferred_element_type=jnp.float32) + be_ref[...]
    C2 = we_ref.shape[1]
    eb = _swish16(e.astype(jnp.bfloat16)).reshape(Ho, Wo, C2)

    ep_ref[1:Ho + 1, :, :] = eb
    epv = ep_ref[...]
    zc2 = jnp.zeros((Ho + 2, 1, C2), jnp.bfloat16)
    w16 = wdw_ref[...]
    taps = []
    for j in range(3):
        if j == 0:
            sj = jnp.concatenate([zc2, epv[:, :Wo - 1, :]], axis=1)
        elif j == 1:
            sj = epv
        else:
            sj = jnp.concatenate([epv[:, 1:, :], zc2], axis=1)
        for i in range(3):
            taps.append(sj[i:i + Ho] * w16[3 * i + j])
    while len(taps) > 1:
        taps = [taps[k] + taps[k + 1] if k + 1 < len(taps) else taps[k]
                for k in range(0, len(taps), 2)]
    y = taps[0].astype(jnp.float32) + bdw_ref[...]
    y = y * jax.nn.sigmoid(y)

    pooled = jnp.sum(jnp.sum(y, axis=0), axis=0, keepdims=True) * inv_s
    r = jnp.dot(pooled.astype(jnp.bfloat16), w1_ref[...],
                preferred_element_type=jnp.float32) + b1_ref[...]
    r = r * jax.nn.sigmoid(r)
    g = jax.nn.sigmoid(jnp.dot(r.astype(jnp.bfloat16), w2_ref[...],
                               preferred_element_type=jnp.float32)
                       + b2_ref[...])

    gb = g.astype(jnp.bfloat16)
    dg = y.astype(jnp.bfloat16).reshape(S, C2) * gb
    hn = (jnp.dot(dg, wp_ref[...], preferred_element_type=jnp.float32)
          + bp_ref[...] + hb.astype(jnp.float32))
    hd = (jnp.dot(hn.astype(jnp.bfloat16), wh_ref[...],
                  preferred_element_type=jnp.float32) + bh_ref[...])
    hd = hd * jax.nn.sigmoid(hd)
    p2 = jnp.sum(hd, axis=0, keepdims=True) * inv_s
    logits = (jnp.dot(p2.astype(jnp.bfloat16), wf_ref[...],
                      preferred_element_type=jnp.float32) + bf_ref[...])
    o_ref[0] = logits


def _blockdiag2(w):
    K, N = w.shape
    z = jnp.zeros((K, N), w.dtype)
    return jnp.concatenate(
        [jnp.concatenate([w, z], axis=1), jnp.concatenate([z, w], axis=1)],
        axis=0)


def _pair2(v):
    return jnp.tile(v.reshape(1, -1).astype(jnp.float32), (1, 2))


def kernel(x_nchw, stem_w, stem_b, exp_w, exp_b, dw_w, dw_b,
           se_r_w, se_r_b, se_e_w, se_e_b, proj_w, proj_b,
           head_w, head_b, fc_w, fc_b):
    B, C_IN, H, W = x_nchw.shape
    Ho, Wo = H // 2, W // 2
    S = Ho * Wo
    P = B // 2
    Cs = stem_w.shape[1]
    C = exp_w.shape[1]
    Cse = se_r_w.shape[1]
    Ch = head_w.shape[1]
    NC = fc_w.shape[1]

    xr = x_nchw.reshape(P, 2 * C_IN, H, W).astype(jnp.bfloat16)
    cols = jax.lax.conv_general_dilated_patches(
        xr, (3, 3), (2, 2), [(0, 1), (0, 1)],
        dimension_numbers=('NCHW', 'HWIO', 'NHWC')).reshape(P, S,
                                                            9 * 2 * C_IN)

    src = np.zeros(9 * 2 * C_IN, np.int32)
    blk = np.zeros(9 * 2 * C_IN, np.float32)
    for cc in range(2 * C_IN):
        img, corig = divmod(cc, C_IN)
        for t in range(9):
            src[cc * 9 + t] = t * C_IN + corig
            blk[cc * 9 + t] = float(img)
    w54 = stem_w[src]
    m1 = jnp.asarray(blk, stem_w.dtype)[:, None]
    ws2 = jnp.concatenate([w54 * (1.0 - m1), w54 * m1], axis=1)
    we2 = _blockdiag2(exp_w)
    wdw2 = jnp.tile(dw_w, (1, 2))
    w1_2 = _blockdiag2(se_r_w)
    w2_2 = _blockdiag2(se_e_w)
    wp2 = _blockdiag2(proj_w)
    wh2 = _blockdiag2(head_w)
    wf2 = _blockdiag2(fc_w)

    out = pl.pallas_call(
        functools.partial(_fused_net_kernel, Ho=Ho, Wo=Wo),
        out_shape=jax.ShapeDtypeStruct((P, 1, 2 * NC), jnp.float32),
        grid=(P,),
        in_specs=[
            pl.BlockSpec((1, S, 9 * 2 * C_IN), lambda b: (b, 0, 0)),
            pl.BlockSpec((9 * 2 * C_IN, 2 * Cs), lambda b: (0, 0)),
            pl.BlockSpec((1, 2 * Cs), lambda b: (0, 0)),
            pl.BlockSpec((2 * Cs, 2 * C), lambda b: (0, 0)),
            pl.BlockSpec((1, 2 * C), lambda b: (0, 0)),
            pl.BlockSpec((9, 2 * C), lambda b: (0, 0)),
            pl.BlockSpec((1, 2 * C), lambda b: (0, 0)),
            pl.BlockSpec((2 * C, 2 * Cse), lambda b: (0, 0)),
            pl.BlockSpec((1, 2 * Cse), lambda b: (0, 0)),
            pl.BlockSpec((2 * Cse, 2 * C), lambda b: (0, 0)),
            pl.BlockSpec((1, 2 * C), lambda b: (0, 0)),
            pl.BlockSpec((2 * C, 2 * Cs), lambda b: (0, 0)),
            pl.BlockSpec((1, 2 * Cs), lambda b: (0, 0)),
            pl.BlockSpec((2 * Cs, 2 * Ch), lambda b: (0, 0)),
            pl.BlockSpec((1, 2 * Ch), lambda b: (0, 0)),
            pl.BlockSpec((2 * Ch, 2 * NC), lambda b: (0, 0)),
            pl.BlockSpec((1, 2 * NC), lambda b: (0, 0)),
        ],
        out_specs=pl.BlockSpec((1, 1, 2 * NC), lambda b: (b, 0, 0)),
        scratch_shapes=[pltpu.VMEM((Ho + 2, Wo, 2 * C), jnp.bfloat16)],
        compiler_params=pltpu.CompilerParams(
            dimension_semantics=("arbitrary",)),
    )(cols, ws2, _pair2(stem_b), we2, _pair2(exp_b),
      wdw2, _pair2(dw_b), w1_2, _pair2(se_r_b), w2_2, _pair2(se_e_b),
      wp2, _pair2(proj_b), wh2, _pair2(head_b), wf2, _pair2(fc_b))
    return out.reshape(P, 2, NC).reshape(B, NC)
